# baseline (speedup 1.0000x reference)
import functools

import jax
import jax.numpy as jnp
from jax import lax
from jax.experimental import pallas as pl
from jax.experimental.pallas import tpu as pltpu
from jax.experimental.pallas import tpu_sc as plsc

BATCH = 16384
N_FIELDS = 4
DIM = 16

_NC = 2
_NS = 16
_NW = _NC * _NS
_BPW = BATCH // _NW
_CHUNK = 128
_NCHUNK = _BPW // _CHUNK

_mesh = plsc.VectorSubcoreMesh(core_axis_name="c", subcore_axis_name="s")


@functools.partial(
    pl.kernel,
    mesh=_mesh,
    compiler_params=pltpu.CompilerParams(use_tc_tiling_on_sc=False),
    out_type=jax.ShapeDtypeStruct((BATCH, N_FIELDS, DIM), jnp.float32),
    scratch_types=[
        pltpu.VMEM((N_FIELDS, _BPW), jnp.int32),
        pltpu.VMEM((N_FIELDS, _BPW, DIM), jnp.float32),
        pltpu.SemaphoreType.DMA,
    ],
)
def _emb_lookup(xT_hbm, w0, w1, w2, w3, out_hbm, idx_v, rows_v, sem):
    wid = lax.axis_index("s") * _NC + lax.axis_index("c")
    base = wid * _BPW
    pltpu.sync_copy(xT_hbm.at[:, pl.ds(base, _BPW)], idx_v)
    tables = [w0, w1, w2, w3]
    handles = []
    for f in range(N_FIELDS):
        for c in range(_NCHUNK):
            handles.append(
                pltpu.async_copy(
                    tables[f].at[idx_v.at[f, pl.ds(c * _CHUNK, _CHUNK)]],
                    rows_v.at[f, pl.ds(c * _CHUNK, _CHUNK)],
                    sem,
                )
            )
    for h in handles:
        h.wait()
    for f in range(N_FIELDS):
        pltpu.sync_copy(rows_v.at[f], out_hbm.at[pl.ds(base, _BPW), f])


def kernel(x, W0, W1, W2, W3):
    xT = x.astype(jnp.int32).T
    out = _emb_lookup(xT, W0, W1, W2, W3)
    return out.reshape(BATCH, N_FIELDS * DIM)
